# baseline (speedup 1.0000x reference)
import jax
import jax.numpy as jnp
from jax import lax
from jax.experimental import pallas as pl
from jax.experimental.pallas import tpu as pltpu
from jax.experimental.pallas import tpu_sc as plsc

N = 10000
D = 128
E = 320000
L = 2

NC = 2
NS = 16
NP = 10240
EP = 327680
ET = EP // (NC * NS)
EB = ET // 128
RPT = NP // NS
PAD_IDX = N

def _mesh():
    return plsc.VectorSubcoreMesh(
        core_axis_name="c", subcore_axis_name="s", num_cores=NC, num_subcores=NS
    )


def _deg_body(src_hbm, out_hbm, idx_v, ones_v, acc_sh):
    c = lax.axis_index("c")
    s = lax.axis_index("s")
    t = c * NS + s
    @pl.loop(0, 128)
    def _(i):
        @pl.loop(0, D, step=16)
        def _(j):
            ones_v[i, pl.ds(j, 16)] = jnp.zeros((16,), jnp.float32)

    @pl.loop(0, RPT, step=128)
    def _(r):
        pltpu.sync_copy(ones_v, acc_sh.at[pl.ds(s * RPT + r, 128)])

    @pl.loop(0, 128)
    def _(i):
        @pl.loop(0, D, step=16)
        def _(j):
            ones_v[i, pl.ds(j, 16)] = jnp.full((16,), 1.0, jnp.float32)

    plsc.subcore_barrier()
    pltpu.sync_copy(src_hbm.at[pl.ds(t * EB, EB)], idx_v)

    @pl.loop(0, EB)
    def _(j):
        pltpu.sync_copy(ones_v, acc_sh.at[idx_v.at[j]], add=True)

    plsc.subcore_barrier()
    pltpu.sync_copy(
        acc_sh.at[pl.ds(s * RPT, RPT)], out_hbm.at[pl.ds(c * NP + s * RPT, RPT)]
    )


def _sc_deg(srcT):
    return pl.kernel(
        _deg_body,
        out_type=jax.ShapeDtypeStruct((NC * NP, D), jnp.float32),
        mesh=_mesh(),
        scratch_types=[
            pltpu.VMEM((EB, 128), jnp.int32),
            pltpu.VMEM((128, D), jnp.float32),
            pltpu.VMEM_SHARED((NP, D), jnp.float32),
        ],
    )(srcT)


_CB = 16
_SUBG = 4
_EBF = 128
_EBS = 32
TOTB = EP // 128


def _segsum_body(g_hbm, src_hbm, dst_hbm, out_hbm, sA, sB, dA, dB,
                 r0, r1, acc_sh, g0, g1, s0, s1, isem):
    c = lax.axis_index("c")
    s = lax.axis_index("s")
    rows = [r0, r1]
    gsem = [g0, g1]
    ssem = [s0, s1]
    sidx = [sA, sB]
    didx = [dA, dB]

    @pl.loop(0, 128)
    def _(i):
        @pl.loop(0, D, step=16)
        def _(j):
            r0[i, pl.ds(j, 16)] = jnp.zeros((16,), jnp.float32)

    @pl.loop(0, RPT, step=128)
    def _(r):
        pltpu.sync_copy(r0, acc_sh.at[pl.ds(s * RPT + r, 128)])

    plsc.subcore_barrier()

    def start_gather(si, blk, slot):
        sz = 128 // _SUBG
        for u in range(_SUBG):
            pltpu.async_copy(
                g_hbm.at[si.at[blk, pl.ds(u * sz, sz)]],
                rows[slot].at[pl.ds(u * sz, sz)],
                gsem[slot],
            )

    def wait_sem(sem, slot):
        pltpu.make_async_copy(g_hbm.at[pl.ds(0, 128)], rows[slot], sem).wait()

    def start_scatter(di, blk, slot):
        pltpu.async_copy(rows[slot], acc_sh.at[di.at[blk]], ssem[slot], add=True)

    def pipeline(blk_base, nchk):
        def start_idx_load(ci, pi):
            base = blk_base + ci * _CB
            pltpu.async_copy(src_hbm.at[pl.ds(base, _CB)], sidx[pi], isem)
            pltpu.async_copy(dst_hbm.at[pl.ds(base, _CB)], didx[pi], isem)

        def drain_idx_loads(pi):
            pltpu.make_async_copy(src_hbm.at[pl.ds(0, _CB)], sidx[pi], isem).wait()
            pltpu.make_async_copy(src_hbm.at[pl.ds(0, _CB)], didx[pi], isem).wait()

        start_idx_load(0, 0)
        drain_idx_loads(0)
        for ci in range(nchk):
            pi = ci % 2
            si, di = sidx[pi], didx[pi]
            start_gather(si, 0, 0)
            wait_sem(gsem[0], 0)
            start_scatter(di, 0, 0)
            if ci > 0:
                wait_sem(ssem[1], 1)
            if ci + 1 < nchk:
                start_idx_load(ci + 1, 1 - pi)
            start_gather(si, 1, 1)

            @pl.loop(0, (_CB - 2) // 2)
            def _(m):
                for k in range(2):
                    q = 1 + 2 * m + k
                    slot = (1 + k) % 2
                    wait_sem(gsem[slot], slot)
                    start_scatter(di, q, slot)
                    wait_sem(ssem[1 - slot], 1 - slot)
                    start_gather(si, q + 1, 1 - slot)

            wait_sem(gsem[1], 1)
            start_scatter(di, _CB - 1, 1)
            wait_sem(ssem[0], 0)
            if ci + 1 < nchk:
                drain_idx_loads(1 - pi)

        wait_sem(ssem[1], 1)

    @pl.when(c == 0)
    def _():
        pipeline(s * _EBF, _EBF // _CB)

    @pl.when(c == 1)
    def _():
        pipeline(NS * _EBF + s * _EBS, _EBS // _CB)

    plsc.subcore_barrier()
    pltpu.sync_copy(
        acc_sh.at[pl.ds(s * RPT, RPT)], out_hbm.at[pl.ds(c * NP + s * RPT, RPT)]
    )


def _sc_segsum(g, srcT, dstT):
    return pl.kernel(
        _segsum_body,
        out_type=jax.ShapeDtypeStruct((NC * NP, D), jnp.float32),
        mesh=_mesh(),
        scratch_types=[
            pltpu.VMEM((_CB, 128), jnp.int32),
            pltpu.VMEM((_CB, 128), jnp.int32),
            pltpu.VMEM((_CB, 128), jnp.int32),
            pltpu.VMEM((_CB, 128), jnp.int32),
            pltpu.VMEM((128, D), jnp.float32),
            pltpu.VMEM((128, D), jnp.float32),
            pltpu.VMEM_SHARED((NP, D), jnp.float32),
            pltpu.SemaphoreType.DMA,
            pltpu.SemaphoreType.DMA,
            pltpu.SemaphoreType.DMA,
            pltpu.SemaphoreType.DMA,
            pltpu.SemaphoreType.DMA,
        ],
    )(g, srcT, dstT)


def _dis(deg2_ref):
    d = deg2_ref[0:N, 0:1] + deg2_ref[NP : NP + N, 0:1]
    return jnp.where(d > 0, lax.rsqrt(d), 0.0)


def _gelu(x):
    return x * 0.5 * (1.0 + lax.erf(x * 0.7071067811865476))


def _prep_body(deg2_ref, x_ref, g_ref):
    dis = _dis(deg2_ref)
    g_ref[0:N, :] = x_ref[...] * dis
    g_ref[N:NP, :] = jnp.zeros((NP - N, D), jnp.float32)


def _tc_prep(deg2, x):
    return pl.pallas_call(
        _prep_body,
        out_shape=jax.ShapeDtypeStruct((NP, D), jnp.float32),
    )(deg2, x)


def _mid_body(sp_ref, deg2_ref, tx1_ref, g1_ref):
    dis = _dis(deg2_ref)
    ssum = sp_ref[0:N, :] + sp_ref[NP : NP + N, :]
    tx1 = -dis * ssum
    tx1_ref[...] = tx1
    g1_ref[0:N, :] = dis * tx1
    g1_ref[N:NP, :] = jnp.zeros((NP - N, D), jnp.float32)


def _tc_mid(sparts, deg2):
    return pl.pallas_call(
        _mid_body,
        out_shape=(
            jax.ShapeDtypeStruct((N, D), jnp.float32),
            jax.ShapeDtypeStruct((NP, D), jnp.float32),
        ),
    )(sparts, deg2)


def _layer_body(h_ref, tx1_ref, qp_ref, deg2_ref, w_ref, b_ref, gam_ref, bet_ref,
                out_ref, gn_ref):
    dis = _dis(deg2_ref)
    h = h_ref[...]
    tx1 = tx1_ref[...]
    q = qp_ref[0:N, :] + qp_ref[NP : NP + N, :]
    tx2 = -2.0 * dis * q - h
    acc = jnp.dot(h, w_ref[0], preferred_element_type=jnp.float32)
    acc = acc + jnp.dot(tx1, w_ref[1], preferred_element_type=jnp.float32)
    acc = acc + jnp.dot(tx2, w_ref[2], preferred_element_type=jnp.float32)
    acc = acc + b_ref[...]
    mu = jnp.mean(acc, axis=0, keepdims=True)
    var = jnp.mean(acc * acc, axis=0, keepdims=True) - mu * mu
    acc = (acc - mu) * lax.rsqrt(var + 1e-5) * gam_ref[...] + bet_ref[...]
    out = _gelu(acc + h)
    out_ref[...] = out
    gn_ref[0:N, :] = dis * out
    gn_ref[N:NP, :] = jnp.zeros((NP - N, D), jnp.float32)


def _tc_layer(h, tx1, qparts, deg2, w, b, gam, bet):
    return pl.pallas_call(
        _layer_body,
        out_shape=(
            jax.ShapeDtypeStruct((N, D), jnp.float32),
            jax.ShapeDtypeStruct((NP, D), jnp.float32),
        ),
    )(h, tx1, qparts, deg2, w, b, gam, bet)


def _mlp_body(h_ref, w1_ref, b1_ref, w2_ref, b2_ref, y_ref):
    hid = _gelu(
        jnp.dot(h_ref[...], w1_ref[...], preferred_element_type=jnp.float32)
        + b1_ref[...]
    )
    y_ref[...] = (
        jnp.dot(hid, w2_ref[...], preferred_element_type=jnp.float32) + b2_ref[...]
    )


def _tc_mlp(h, w1, b1, w2, b2):
    return pl.pallas_call(
        _mlp_body,
        out_shape=jax.ShapeDtypeStruct((N, D), jnp.float32),
    )(h, w1, b1, w2, b2)


def kernel(x, edge_index, W_cheb, b_cheb, gamma, beta, W1, b1, W2, b2):
    src = edge_index[0]
    dst = edge_index[1]
    pad = jnp.full((EP - E,), PAD_IDX, jnp.int32)
    srcT = jnp.concatenate([src, pad]).reshape(EP // 128, 128)
    dstT = jnp.concatenate([dst, pad]).reshape(EP // 128, 128)

    deg2 = _sc_deg(srcT)
    h = x
    g = _tc_prep(deg2, x)
    for l in range(L):
        sparts = _sc_segsum(g, srcT, dstT)
        tx1, g1 = _tc_mid(sparts, deg2)
        qparts = _sc_segsum(g1, srcT, dstT)
        h, g = _tc_layer(
            h, tx1, qparts, deg2,
            W_cheb[l], b_cheb[l].reshape(1, D),
            gamma[l].reshape(1, D), beta[l].reshape(1, D),
        )
    return _tc_mlp(h, W1, b1.reshape(1, D), W2, b2.reshape(1, D))


# speedup vs baseline: 1.0474x; 1.0474x over previous
import jax
import jax.numpy as jnp
from jax import lax
from jax.experimental import pallas as pl
from jax.experimental.pallas import tpu as pltpu
from jax.experimental.pallas import tpu_sc as plsc

N = 10000
D = 128
E = 320000
L = 2

NC = 2
NS = 16
NP = 10240
EP = 327680
ET = EP // (NC * NS)
EB = ET // 128
RPT = NP // NS
PAD_IDX = N

def _mesh():
    return plsc.VectorSubcoreMesh(
        core_axis_name="c", subcore_axis_name="s", num_cores=NC, num_subcores=NS
    )


def _deg_body(src_hbm, out_hbm, idx_v, ones_v, acc_sh):
    c = lax.axis_index("c")
    s = lax.axis_index("s")
    t = c * NS + s
    @pl.loop(0, 128)
    def _(i):
        @pl.loop(0, D, step=16)
        def _(j):
            ones_v[i, pl.ds(j, 16)] = jnp.zeros((16,), jnp.float32)

    @pl.loop(0, RPT, step=128)
    def _(r):
        pltpu.sync_copy(ones_v, acc_sh.at[pl.ds(s * RPT + r, 128)])

    @pl.loop(0, 128)
    def _(i):
        @pl.loop(0, D, step=16)
        def _(j):
            ones_v[i, pl.ds(j, 16)] = jnp.full((16,), 1.0, jnp.float32)

    plsc.subcore_barrier()
    pltpu.sync_copy(src_hbm.at[pl.ds(t * EB, EB)], idx_v)

    @pl.loop(0, EB)
    def _(j):
        pltpu.sync_copy(ones_v, acc_sh.at[idx_v.at[j]], add=True)

    plsc.subcore_barrier()
    pltpu.sync_copy(
        acc_sh.at[pl.ds(s * RPT, RPT)], out_hbm.at[pl.ds(c * NP + s * RPT, RPT)]
    )


def _sc_deg(srcT):
    return pl.kernel(
        _deg_body,
        out_type=jax.ShapeDtypeStruct((NC * NP, D), jnp.float32),
        mesh=_mesh(),
        scratch_types=[
            pltpu.VMEM((EB, 128), jnp.int32),
            pltpu.VMEM((128, D), jnp.float32),
            pltpu.VMEM_SHARED((NP, D), jnp.float32),
        ],
    )(srcT)


_CB = 16
_SUBG = 4
_EBF = 144
_EBS = 16
TOTB = EP // 128


def _segsum_body(g_hbm, src_hbm, dst_hbm, out_hbm, sA, sB, dA, dB,
                 r0, r1, acc_sh, g0, g1, s0, s1, isem):
    c = lax.axis_index("c")
    s = lax.axis_index("s")
    rows = [r0, r1]
    gsem = [g0, g1]
    ssem = [s0, s1]
    sidx = [sA, sB]
    didx = [dA, dB]

    @pl.loop(0, 128)
    def _(i):
        @pl.loop(0, D, step=16)
        def _(j):
            r0[i, pl.ds(j, 16)] = jnp.zeros((16,), jnp.float32)

    @pl.loop(0, RPT, step=128)
    def _(r):
        pltpu.sync_copy(r0, acc_sh.at[pl.ds(s * RPT + r, 128)])

    plsc.subcore_barrier()

    def start_gather(si, blk, slot):
        sz = 128 // _SUBG
        for u in range(_SUBG):
            pltpu.async_copy(
                g_hbm.at[si.at[blk, pl.ds(u * sz, sz)]],
                rows[slot].at[pl.ds(u * sz, sz)],
                gsem[slot],
            )

    def wait_sem(sem, slot):
        pltpu.make_async_copy(g_hbm.at[pl.ds(0, 128)], rows[slot], sem).wait()

    def start_scatter(di, blk, slot):
        pltpu.async_copy(rows[slot], acc_sh.at[di.at[blk]], ssem[slot], add=True)

    def pipeline(blk_base, nchk):
        def start_idx_load(ci, pi):
            base = blk_base + ci * _CB
            pltpu.async_copy(src_hbm.at[pl.ds(base, _CB)], sidx[pi], isem)
            pltpu.async_copy(dst_hbm.at[pl.ds(base, _CB)], didx[pi], isem)

        def drain_idx_loads(pi):
            pltpu.make_async_copy(src_hbm.at[pl.ds(0, _CB)], sidx[pi], isem).wait()
            pltpu.make_async_copy(src_hbm.at[pl.ds(0, _CB)], didx[pi], isem).wait()

        start_idx_load(0, 0)
        drain_idx_loads(0)
        for ci in range(nchk):
            pi = ci % 2
            si, di = sidx[pi], didx[pi]
            start_gather(si, 0, 0)
            wait_sem(gsem[0], 0)
            start_scatter(di, 0, 0)
            if ci > 0:
                wait_sem(ssem[1], 1)
            if ci + 1 < nchk:
                start_idx_load(ci + 1, 1 - pi)
            start_gather(si, 1, 1)

            @pl.loop(0, (_CB - 2) // 2)
            def _(m):
                for k in range(2):
                    q = 1 + 2 * m + k
                    slot = (1 + k) % 2
                    wait_sem(gsem[slot], slot)
                    start_scatter(di, q, slot)
                    wait_sem(ssem[1 - slot], 1 - slot)
                    start_gather(si, q + 1, 1 - slot)

            wait_sem(gsem[1], 1)
            start_scatter(di, _CB - 1, 1)
            wait_sem(ssem[0], 0)
            if ci + 1 < nchk:
                drain_idx_loads(1 - pi)

        wait_sem(ssem[1], 1)

    @pl.when(c == 0)
    def _():
        pipeline(s * _EBF, _EBF // _CB)

    @pl.when(c == 1)
    def _():
        pipeline(NS * _EBF + s * _EBS, _EBS // _CB)

    plsc.subcore_barrier()
    pltpu.sync_copy(
        acc_sh.at[pl.ds(s * RPT, RPT)], out_hbm.at[pl.ds(c * NP + s * RPT, RPT)]
    )


def _sc_segsum(g, srcT, dstT):
    return pl.kernel(
        _segsum_body,
        out_type=jax.ShapeDtypeStruct((NC * NP, D), jnp.float32),
        mesh=_mesh(),
        scratch_types=[
            pltpu.VMEM((_CB, 128), jnp.int32),
            pltpu.VMEM((_CB, 128), jnp.int32),
            pltpu.VMEM((_CB, 128), jnp.int32),
            pltpu.VMEM((_CB, 128), jnp.int32),
            pltpu.VMEM((128, D), jnp.float32),
            pltpu.VMEM((128, D), jnp.float32),
            pltpu.VMEM_SHARED((NP, D), jnp.float32),
            pltpu.SemaphoreType.DMA,
            pltpu.SemaphoreType.DMA,
            pltpu.SemaphoreType.DMA,
            pltpu.SemaphoreType.DMA,
            pltpu.SemaphoreType.DMA,
        ],
    )(g, srcT, dstT)


def _dis(deg2_ref):
    d = deg2_ref[0:N, 0:1] + deg2_ref[NP : NP + N, 0:1]
    return jnp.where(d > 0, lax.rsqrt(d), 0.0)


def _gelu(x):
    return x * 0.5 * (1.0 + lax.erf(x * 0.7071067811865476))


def _prep_body(deg2_ref, x_ref, g_ref):
    dis = _dis(deg2_ref)
    g_ref[0:N, :] = x_ref[...] * dis
    g_ref[N:NP, :] = jnp.zeros((NP - N, D), jnp.float32)


def _tc_prep(deg2, x):
    return pl.pallas_call(
        _prep_body,
        out_shape=jax.ShapeDtypeStruct((NP, D), jnp.float32),
    )(deg2, x)


def _mid_body(sp_ref, deg2_ref, tx1_ref, g1_ref):
    dis = _dis(deg2_ref)
    ssum = sp_ref[0:N, :] + sp_ref[NP : NP + N, :]
    tx1 = -dis * ssum
    tx1_ref[...] = tx1
    g1_ref[0:N, :] = dis * tx1
    g1_ref[N:NP, :] = jnp.zeros((NP - N, D), jnp.float32)


def _tc_mid(sparts, deg2):
    return pl.pallas_call(
        _mid_body,
        out_shape=(
            jax.ShapeDtypeStruct((N, D), jnp.float32),
            jax.ShapeDtypeStruct((NP, D), jnp.float32),
        ),
    )(sparts, deg2)


def _layer_body(h_ref, tx1_ref, qp_ref, deg2_ref, w_ref, b_ref, gam_ref, bet_ref,
                out_ref, gn_ref):
    dis = _dis(deg2_ref)
    h = h_ref[...]
    tx1 = tx1_ref[...]
    q = qp_ref[0:N, :] + qp_ref[NP : NP + N, :]
    tx2 = -2.0 * dis * q - h
    acc = jnp.dot(h, w_ref[0], preferred_element_type=jnp.float32)
    acc = acc + jnp.dot(tx1, w_ref[1], preferred_element_type=jnp.float32)
    acc = acc + jnp.dot(tx2, w_ref[2], preferred_element_type=jnp.float32)
    acc = acc + b_ref[...]
    mu = jnp.mean(acc, axis=0, keepdims=True)
    var = jnp.mean(acc * acc, axis=0, keepdims=True) - mu * mu
    acc = (acc - mu) * lax.rsqrt(var + 1e-5) * gam_ref[...] + bet_ref[...]
    out = _gelu(acc + h)
    out_ref[...] = out
    gn_ref[0:N, :] = dis * out
    gn_ref[N:NP, :] = jnp.zeros((NP - N, D), jnp.float32)


def _tc_layer(h, tx1, qparts, deg2, w, b, gam, bet):
    return pl.pallas_call(
        _layer_body,
        out_shape=(
            jax.ShapeDtypeStruct((N, D), jnp.float32),
            jax.ShapeDtypeStruct((NP, D), jnp.float32),
        ),
    )(h, tx1, qparts, deg2, w, b, gam, bet)


def _mlp_body(h_ref, w1_ref, b1_ref, w2_ref, b2_ref, y_ref):
    hid = _gelu(
        jnp.dot(h_ref[...], w1_ref[...], preferred_element_type=jnp.float32)
        + b1_ref[...]
    )
    y_ref[...] = (
        jnp.dot(hid, w2_ref[...], preferred_element_type=jnp.float32) + b2_ref[...]
    )


def _tc_mlp(h, w1, b1, w2, b2):
    return pl.pallas_call(
        _mlp_body,
        out_shape=jax.ShapeDtypeStruct((N, D), jnp.float32),
    )(h, w1, b1, w2, b2)


def kernel(x, edge_index, W_cheb, b_cheb, gamma, beta, W1, b1, W2, b2):
    src = edge_index[0]
    dst = edge_index[1]
    pad = jnp.full((EP - E,), PAD_IDX, jnp.int32)
    srcT = jnp.concatenate([src, pad]).reshape(EP // 128, 128)
    dstT = jnp.concatenate([dst, pad]).reshape(EP // 128, 128)

    deg2 = _sc_deg(srcT)
    h = x
    g = _tc_prep(deg2, x)
    for l in range(L):
        sparts = _sc_segsum(g, srcT, dstT)
        tx1, g1 = _tc_mid(sparts, deg2)
        qparts = _sc_segsum(g1, srcT, dstT)
        h, g = _tc_layer(
            h, tx1, qparts, deg2,
            W_cheb[l], b_cheb[l].reshape(1, D),
            gamma[l].reshape(1, D), beta[l].reshape(1, D),
        )
    return _tc_mlp(h, W1, b1.reshape(1, D), W2, b2.reshape(1, D))
